# Initial kernel scaffold; baseline (speedup 1.0000x reference)
import jax
import jax.numpy as jnp
from jax.experimental import pallas as pl


def kernel(features, labels, center):
    raise NotImplementedError("write your pallas kernel here")


# speedup vs baseline: 2.7831x; 2.7831x over previous
import jax
import jax.numpy as jnp
from jax.experimental import pallas as pl
from jax.experimental.pallas import tpu as pltpu

NUM_CLASS = 1000
CPAD = 1024
FEAT_DIM = 512
BATCH = 32768
THRESHOLD = 5.0

NCORE = 2
BBLK = 1024
NB = BATCH // (NCORE * BBLK)


def _seg_kernel(feat_ref, lab_ref, sum_ref, cnt_ref):
    j = pl.program_id(1)
    lab = lab_ref[0, 0, :]
    cls = jax.lax.broadcasted_iota(jnp.int32, (CPAD, BBLK), 0)
    onehot = jnp.where(lab[None, :] == cls, 1.0, 0.0)
    oh = onehot.astype(jnp.bfloat16)

    feat = feat_ref[...]
    hi = feat.astype(jnp.bfloat16)
    lo = (feat - hi.astype(jnp.float32)).astype(jnp.bfloat16)

    psum = jnp.dot(oh, hi, preferred_element_type=jnp.float32)
    psum += jnp.dot(oh, lo, preferred_element_type=jnp.float32)
    ones = jnp.ones((BBLK, 128), dtype=jnp.bfloat16)
    pcnt = jnp.dot(oh, ones, preferred_element_type=jnp.float32)

    @pl.when(j == 0)
    def _():
        sum_ref[0] = psum
        cnt_ref[0] = pcnt

    @pl.when(j > 0)
    def _():
        sum_ref[0] += psum
        cnt_ref[0] += pcnt


def _dist_kernel(sum_h_ref, cnt_h_ref, cen_h_ref,
                 sum_f_ref, cnt_f_ref, cen_f_ref,
                 nc_ref, loss_ref):
    i = pl.program_id(0)
    half = CPAD // NCORE

    cnt_f = cnt_f_ref[0] + cnt_f_ref[1]
    recip_f = 1.0 / jnp.maximum(cnt_f, 1.0)
    sums_f = sum_f_ref[0] + sum_f_ref[1]
    nc = cen_f_ref[...] + sums_f * pltpu.repeat(recip_f, FEAT_DIM // 128, axis=1)

    cnt_h = cnt_h_ref[0] + cnt_h_ref[1]
    recip_h = 1.0 / jnp.maximum(cnt_h, 1.0)
    sums_h = sum_h_ref[0] + sum_h_ref[1]
    nc_h = cen_h_ref[...] + sums_h * pltpu.repeat(recip_h, FEAT_DIM // 128, axis=1)
    nc_ref[...] = nc_h

    nc_b = nc.astype(jnp.bfloat16)
    nch_b = nc_h.astype(jnp.bfloat16)
    gram = jax.lax.dot_general(
        nch_b, nc_b, (((1,), (1,)), ((), ())),
        preferred_element_type=jnp.float32)

    sq_h = jnp.sum(nc_h * nc_h, axis=1, keepdims=True)
    nc2 = nc * nc
    nc2_hi = nc2.astype(jnp.bfloat16)
    nc2_lo = (nc2 - nc2_hi.astype(jnp.float32)).astype(jnp.bfloat16)
    ones8 = jnp.ones((8, FEAT_DIM), dtype=jnp.bfloat16)
    sq_row = jax.lax.dot_general(
        ones8, nc2_hi, (((1,), (1,)), ((), ())),
        preferred_element_type=jnp.float32)
    sq_row += jax.lax.dot_general(
        ones8, nc2_lo, (((1,), (1,)), ((), ())),
        preferred_element_type=jnp.float32)

    d2 = sq_h + sq_row[0:1, :] - 2.0 * gram
    dist = jnp.sqrt(jnp.maximum(d2, 0.0))
    hinge = jnp.where(dist < THRESHOLD, THRESHOLD - dist, 0.0)

    rows = jax.lax.broadcasted_iota(jnp.int32, (half, 1), 0) + i * half
    cols = jax.lax.broadcasted_iota(jnp.int32, (1, CPAD), 1)
    rmask = jnp.where(rows < NUM_CLASS, 1.0, 0.0)
    cmask = jnp.where(cols < NUM_CLASS, 1.0, 0.0)
    hinge = hinge * rmask * cmask

    scale = 1.0 / (NUM_CLASS * NUM_CLASS)
    loss_ref[0] = jnp.sum(hinge, keepdims=True) * scale


def kernel(features, labels, center):
    labels = labels.astype(jnp.int32).reshape(NCORE * NB, 1, BBLK)

    psums, pcnts = pl.pallas_call(
        _seg_kernel,
        grid=(NCORE, NB),
        in_specs=[
            pl.BlockSpec((BBLK, FEAT_DIM), lambda i, j: (i * NB + j, 0)),
            pl.BlockSpec((1, 1, BBLK), lambda i, j: (i * NB + j, 0, 0)),
        ],
        out_specs=[
            pl.BlockSpec((1, CPAD, FEAT_DIM), lambda i, j: (i, 0, 0)),
            pl.BlockSpec((1, CPAD, 128), lambda i, j: (i, 0, 0)),
        ],
        out_shape=[
            jax.ShapeDtypeStruct((NCORE, CPAD, FEAT_DIM), jnp.float32),
            jax.ShapeDtypeStruct((NCORE, CPAD, 128), jnp.float32),
        ],
        compiler_params=pltpu.CompilerParams(
            dimension_semantics=("parallel", "arbitrary")),
    )(features, labels)

    cen_pad = jnp.pad(center, ((0, CPAD - NUM_CLASS), (0, 0)))
    half = CPAD // NCORE

    nc_pad, lparts = pl.pallas_call(
        _dist_kernel,
        grid=(NCORE,),
        in_specs=[
            pl.BlockSpec((NCORE, half, FEAT_DIM), lambda i: (0, i, 0)),
            pl.BlockSpec((NCORE, half, 128), lambda i: (0, i, 0)),
            pl.BlockSpec((half, FEAT_DIM), lambda i: (i, 0)),
            pl.BlockSpec((NCORE, CPAD, FEAT_DIM), lambda i: (0, 0, 0)),
            pl.BlockSpec((NCORE, CPAD, 128), lambda i: (0, 0, 0)),
            pl.BlockSpec((CPAD, FEAT_DIM), lambda i: (0, 0)),
        ],
        out_specs=[
            pl.BlockSpec((half, FEAT_DIM), lambda i: (i, 0)),
            pl.BlockSpec((1, 1, 1), lambda i: (i, 0, 0)),
        ],
        out_shape=[
            jax.ShapeDtypeStruct((CPAD, FEAT_DIM), jnp.float32),
            jax.ShapeDtypeStruct((NCORE, 1, 1), jnp.float32),
        ],
        compiler_params=pltpu.CompilerParams(
            dimension_semantics=("parallel",)),
    )(psums, pcnts, cen_pad, psums, pcnts, cen_pad)

    loss = jnp.sum(lparts)
    return loss, nc_pad[:NUM_CLASS]


# speedup vs baseline: 3.8803x; 1.3943x over previous
import jax
import jax.numpy as jnp
from jax.experimental import pallas as pl
from jax.experimental.pallas import tpu as pltpu

NUM_CLASS = 1000
CPAD = 1024
FEAT_DIM = 512
BATCH = 32768
THRESHOLD = 5.0

NCORE = 2
BBLK = 1024
NB = BATCH // (NCORE * BBLK)


def _seg_kernel(feat_ref, lab_ref, sum_ref, cnt_ref):
    j = pl.program_id(1)
    lab = lab_ref[0, 0, :].astype(jnp.int16)
    cls = jax.lax.broadcasted_iota(jnp.int16, (CPAD, BBLK), 0)
    oh = jnp.where(lab[None, :] == cls,
                   jnp.bfloat16(1.0), jnp.bfloat16(0.0))

    fb = feat_ref[...].astype(jnp.bfloat16)
    psum = jnp.dot(oh, fb, preferred_element_type=jnp.float32)
    ones = jnp.ones((BBLK, 128), dtype=jnp.bfloat16)
    pcnt = jnp.dot(oh, ones, preferred_element_type=jnp.float32)

    @pl.when(j == 0)
    def _():
        sum_ref[0] = psum
        cnt_ref[0] = pcnt

    @pl.when(j > 0)
    def _():
        sum_ref[0] += psum
        cnt_ref[0] += pcnt


def _dist_kernel(sum_h_ref, cnt_h_ref, cen_h_ref,
                 sum_f_ref, cnt_f_ref, cen_f_ref,
                 nc_ref, loss_ref):
    i = pl.program_id(0)
    half = CPAD // NCORE

    cnt_f = cnt_f_ref[0] + cnt_f_ref[1]
    recip_f = 1.0 / jnp.maximum(cnt_f, 1.0)
    sums_f = sum_f_ref[0] + sum_f_ref[1]
    nc = cen_f_ref[...] + sums_f * pltpu.repeat(recip_f, FEAT_DIM // 128, axis=1)

    cnt_h = cnt_h_ref[0] + cnt_h_ref[1]
    recip_h = 1.0 / jnp.maximum(cnt_h, 1.0)
    sums_h = sum_h_ref[0] + sum_h_ref[1]
    nc_h = cen_h_ref[...] + sums_h * pltpu.repeat(recip_h, FEAT_DIM // 128, axis=1)
    nc_ref[...] = nc_h

    nc_b = nc.astype(jnp.bfloat16)
    nch_b = nc_h.astype(jnp.bfloat16)
    gram = jax.lax.dot_general(
        nch_b, nc_b, (((1,), (1,)), ((), ())),
        preferred_element_type=jnp.float32)

    sq_h = jnp.sum(nc_h * nc_h, axis=1, keepdims=True)
    nc2 = nc * nc
    nc2_hi = nc2.astype(jnp.bfloat16)
    nc2_lo = (nc2 - nc2_hi.astype(jnp.float32)).astype(jnp.bfloat16)
    ones8 = jnp.ones((8, FEAT_DIM), dtype=jnp.bfloat16)
    sq_row = jax.lax.dot_general(
        ones8, nc2_hi, (((1,), (1,)), ((), ())),
        preferred_element_type=jnp.float32)
    sq_row += jax.lax.dot_general(
        ones8, nc2_lo, (((1,), (1,)), ((), ())),
        preferred_element_type=jnp.float32)

    d2 = sq_h + sq_row[0:1, :] - 2.0 * gram
    dist = jnp.sqrt(jnp.maximum(d2, 0.0))
    hinge = jnp.where(dist < THRESHOLD, THRESHOLD - dist, 0.0)

    rows = jax.lax.broadcasted_iota(jnp.int32, (half, 1), 0) + i * half
    cols = jax.lax.broadcasted_iota(jnp.int32, (1, CPAD), 1)
    rmask = jnp.where(rows < NUM_CLASS, 1.0, 0.0)
    cmask = jnp.where(cols < NUM_CLASS, 1.0, 0.0)
    hinge = hinge * rmask * cmask

    scale = 1.0 / (NUM_CLASS * NUM_CLASS)
    loss_ref[0] = jnp.sum(hinge, keepdims=True) * scale


def kernel(features, labels, center):
    labels = labels.astype(jnp.int32).reshape(NCORE * NB, 1, BBLK)

    psums, pcnts = pl.pallas_call(
        _seg_kernel,
        grid=(NCORE, NB),
        in_specs=[
            pl.BlockSpec((BBLK, FEAT_DIM), lambda i, j: (i * NB + j, 0)),
            pl.BlockSpec((1, 1, BBLK), lambda i, j: (i * NB + j, 0, 0)),
        ],
        out_specs=[
            pl.BlockSpec((1, CPAD, FEAT_DIM), lambda i, j: (i, 0, 0)),
            pl.BlockSpec((1, CPAD, 128), lambda i, j: (i, 0, 0)),
        ],
        out_shape=[
            jax.ShapeDtypeStruct((NCORE, CPAD, FEAT_DIM), jnp.float32),
            jax.ShapeDtypeStruct((NCORE, CPAD, 128), jnp.float32),
        ],
        compiler_params=pltpu.CompilerParams(
            dimension_semantics=("parallel", "arbitrary")),
    )(features, labels)

    cen_pad = jnp.pad(center, ((0, CPAD - NUM_CLASS), (0, 0)))
    half = CPAD // NCORE

    nc_pad, lparts = pl.pallas_call(
        _dist_kernel,
        grid=(NCORE,),
        in_specs=[
            pl.BlockSpec((NCORE, half, FEAT_DIM), lambda i: (0, i, 0)),
            pl.BlockSpec((NCORE, half, 128), lambda i: (0, i, 0)),
            pl.BlockSpec((half, FEAT_DIM), lambda i: (i, 0)),
            pl.BlockSpec((NCORE, CPAD, FEAT_DIM), lambda i: (0, 0, 0)),
            pl.BlockSpec((NCORE, CPAD, 128), lambda i: (0, 0, 0)),
            pl.BlockSpec((CPAD, FEAT_DIM), lambda i: (0, 0)),
        ],
        out_specs=[
            pl.BlockSpec((half, FEAT_DIM), lambda i: (i, 0)),
            pl.BlockSpec((1, 1, 1), lambda i: (i, 0, 0)),
        ],
        out_shape=[
            jax.ShapeDtypeStruct((CPAD, FEAT_DIM), jnp.float32),
            jax.ShapeDtypeStruct((NCORE, 1, 1), jnp.float32),
        ],
        compiler_params=pltpu.CompilerParams(
            dimension_semantics=("parallel",)),
    )(psums, pcnts, cen_pad, psums, pcnts, cen_pad)

    loss = jnp.sum(lparts)
    return loss, nc_pad[:NUM_CLASS]
